# Initial kernel scaffold; baseline (speedup 1.0000x reference)
import jax
import jax.numpy as jnp
from jax.experimental import pallas as pl


def kernel(x, edge_index, edge_attr, W1, b1, W2, b2, W3, b3, W4, b4, W5, b5, W6, b6):
    raise NotImplementedError("write your pallas kernel here")


# speedup vs baseline: 5.5576x; 5.5576x over previous
import functools

import jax
import jax.numpy as jnp
from jax import lax
from jax.experimental import pallas as pl
from jax.experimental.pallas import tpu as pltpu
from jax.experimental.pallas import tpu_sc as plsc

NC = 2
NS = 16
LANES = 16
SUB = 80
CHUNK = 8 * SUB

_mesh = plsc.VectorSubcoreMesh(
    core_axis_name="c", subcore_axis_name="s", num_cores=NC, num_subcores=NS)
_sc_params = pltpu.CompilerParams(needs_layout_passes=False, use_tc_tiling_on_sc=False)


def _row_split(n):
  rpt = (n // NS) // 8 * 8
  last = n - rpt * (NS - 1)
  assert rpt % 8 == 0 and last % 8 == 0
  return rpt, last


def _deg_kernel(n, ep):
  epc = ep // NC
  ept = epc // NS
  NSUB = CHUNK // SUB
  nch = ept // CHUNK
  assert ept % CHUNK == 0

  @functools.partial(
      pl.kernel, mesh=_mesh,
      out_type=[jax.ShapeDtypeStruct((n,), jnp.float32) for _ in range(NC)],
      compiler_params=_sc_params,
      scratch_types=[
          pltpu.VMEM_SHARED((n,), jnp.float32),
          pltpu.VMEM((NSUB, SUB), jnp.int32),
          pltpu.VMEM((CHUNK,), jnp.float32),
      ])
  def deg_k(dst2_hbm, ea_hbm, z1_hbm, deg0_hbm, deg1_hbm, deg_sh, d_v, w_v):
    c = lax.axis_index("c")
    s = lax.axis_index("s")

    @pl.when(s == 0)
    def _():
      pltpu.sync_copy(z1_hbm, deg_sh)
    plsc.subcore_barrier()

    base0 = c * epc + s * ept

    def chunk(k, _):
      base = pl.multiple_of(base0 + k * CHUNK, CHUNK)
      pltpu.sync_copy(dst2_hbm.at[pl.ds(pl.multiple_of(base // SUB, 8), NSUB), :], d_v)
      pltpu.sync_copy(ea_hbm.at[pl.ds(base, CHUNK)], w_v)
      for j in range(NSUB):
        pltpu.sync_copy(w_v.at[pl.ds(j * SUB, SUB)],
                        deg_sh.at[d_v.at[j]], add=True)
      return 0

    lax.fori_loop(0, nch, chunk, 0)
    plsc.subcore_barrier()

    @pl.when(s == 0)
    def _():
      @pl.when(c == 0)
      def _():
        pltpu.sync_copy(deg_sh, deg0_hbm)
      @pl.when(c == 1)
      def _():
        pltpu.sync_copy(deg_sh, deg1_hbm)

  return deg_k


def _norm_kernel(n, ep):
  nw = NC * NS
  ept = ep // nw
  C = 2048
  nch = ept // C
  steps = C // LANES
  assert ept % C == 0

  @functools.partial(
      pl.kernel, mesh=_mesh,
      out_type=jax.ShapeDtypeStruct((ep,), jnp.float32),
      compiler_params=_sc_params,
      scratch_types=[
          pltpu.VMEM((n,), jnp.float32),
          pltpu.VMEM((C,), jnp.int32),
          pltpu.VMEM((C,), jnp.int32),
          pltpu.VMEM((C,), jnp.float32),
          pltpu.VMEM((C,), jnp.float32),
      ])
  def norm_k(src_hbm, dst_hbm, ea_hbm, g_hbm, norm_hbm, g_v, s_v, d_v, w_v, o_v):
    c = lax.axis_index("c")
    s = lax.axis_index("s")
    wid = s * NC + c
    pltpu.sync_copy(g_hbm, g_v)
    base0 = wid * ept

    def chunk(k, _):
      base = pl.multiple_of(base0 + k * C, C)
      pltpu.sync_copy(src_hbm.at[pl.ds(base, C)], s_v)
      pltpu.sync_copy(dst_hbm.at[pl.ds(base, C)], d_v)
      pltpu.sync_copy(ea_hbm.at[pl.ds(base, C)], w_v)

      def step(i, _):
        off = i * LANES
        si = s_v[pl.ds(off, LANES)]
        di = d_v[pl.ds(off, LANES)]
        wv = w_v[pl.ds(off, LANES)]
        o_v[pl.ds(off, LANES)] = (
            plsc.load_gather(g_v, [si]) * wv * plsc.load_gather(g_v, [di]))
        return 0

      lax.fori_loop(0, steps, step, 0)
      pltpu.sync_copy(o_v, norm_hbm.at[pl.ds(base, C)])
      return 0

    lax.fori_loop(0, nch, chunk, 0)

  return norm_k


def _agg_kernel(n, ep, dw):
  ept = ep // NS
  NSUB = CHUNK // SUB
  nch = ept // CHUNK
  rpt, rlast = _row_split(n)
  assert ept % CHUNK == 0 and dw % LANES == 0

  @functools.partial(
      pl.kernel, mesh=_mesh,
      out_type=[jax.ShapeDtypeStruct((n, dw), jnp.float32) for _ in range(NC)],
      compiler_params=_sc_params,
      scratch_types=[
          pltpu.VMEM_SHARED((n, dw), jnp.float32),
          pltpu.VMEM((NSUB, SUB), jnp.int32),
          pltpu.VMEM((NSUB, SUB), jnp.int32),
          pltpu.VMEM((CHUNK,), jnp.float32),
          pltpu.VMEM((CHUNK, dw), jnp.float32),
          pltpu.SemaphoreType.DMA,
      ])
  def agg_k(y0_hbm, y1_hbm, src2_hbm, dst2_hbm, nrm_hbm, z_hbm,
            a0_hbm, a1_hbm, acc_sh, s_v, d_v, n_v, rows_v, sem):
    c = lax.axis_index("c")
    s = lax.axis_index("s")

    @pl.when(s == 0)
    def _():
      pltpu.sync_copy(z_hbm, acc_sh)
    plsc.subcore_barrier()

    base0 = s * ept

    def chunk(k, _):
      base = pl.multiple_of(base0 + k * CHUNK, CHUNK)
      brow = pl.multiple_of(base // SUB, 8)
      pltpu.sync_copy(src2_hbm.at[pl.ds(brow, NSUB), :], s_v)
      pltpu.sync_copy(dst2_hbm.at[pl.ds(brow, NSUB), :], d_v)
      pltpu.sync_copy(nrm_hbm.at[pl.ds(base, CHUNK)], n_v)

      for j in range(NSUB):
        rows_j = rows_v.at[pl.ds(j * SUB, SUB), :]
        @pl.when(c == 0)
        def _():
          pltpu.async_copy(y0_hbm.at[s_v.at[j]], rows_j, sem).wait()
        @pl.when(c == 1)
        def _():
          pltpu.async_copy(y1_hbm.at[s_v.at[j]], rows_j, sem).wait()

      def srow(r, _):
        spl = plsc.load_gather(n_v, [jnp.full((LANES,), 0, jnp.int32) + r])
        for jj in range(dw // LANES):
          sl = pl.ds(jj * LANES, LANES)
          rows_v[r, sl] = rows_v[r, sl] * spl
        return 0

      lax.fori_loop(0, CHUNK, srow, 0)

      for j in range(NSUB):
        pltpu.sync_copy(rows_v.at[pl.ds(j * SUB, SUB), :],
                        acc_sh.at[d_v.at[j]], add=True)
      return 0

    lax.fori_loop(0, nch, chunk, 0)
    plsc.subcore_barrier()

    r0 = pl.multiple_of(s * rpt, 8)
    for cc, a_hbm in ((0, a0_hbm), (1, a1_hbm)):
      @pl.when((c == cc) & (s < NS - 1))
      def _():
        pltpu.sync_copy(acc_sh.at[pl.ds(r0, rpt), :],
                        a_hbm.at[pl.ds(r0, rpt), :])
      @pl.when((c == cc) & (s == NS - 1))
      def _():
        pltpu.sync_copy(acc_sh.at[pl.ds(r0, rlast), :],
                        a_hbm.at[pl.ds(r0, rlast), :])

  return agg_k


def _prep_kernel(r, q):
  def body(d0_ref, d1_ref, g_ref, g2_ref):
    deg = d0_ref[...] + d1_ref[...] + 1.0
    g = jnp.where(deg > 0, lax.rsqrt(deg), 0.0)
    g_ref[...] = g
    g2_ref[...] = g * g

  return pl.pallas_call(
      body,
      out_shape=[jax.ShapeDtypeStruct((r, q), jnp.float32)] * 2)


def _mm_first(n, din, dq):
  B = 1000
  dh = dq // 2

  def body(x_ref, w_ref, y0_ref, y1_ref):
    y = jnp.dot(x_ref[...], w_ref[...], preferred_element_type=jnp.float32)
    y0_ref[...] = y[:, :dh]
    y1_ref[...] = y[:, dh:]

  return pl.pallas_call(
      body,
      grid=(n // B,),
      in_specs=[pl.BlockSpec((B, din), lambda i: (i, 0)),
                pl.BlockSpec((din, dq), lambda i: (0, 0))],
      out_specs=[pl.BlockSpec((B, dh), lambda i: (i, 0))] * 2,
      out_shape=[jax.ShapeDtypeStruct((n, dh), jnp.float32)] * 2)


def _mm_mid(n, dp, dq, act):
  B = 1000
  dhp = dp // 2
  dhq = dq // 2

  def body(a0_ref, a1_ref, y0_ref, y1_ref, g2_ref, b_ref, w_ref,
           o0_ref, o1_ref):
    agg = jnp.concatenate([a0_ref[...], a1_ref[...]], axis=1)
    y = jnp.concatenate([y0_ref[...], y1_ref[...]], axis=1)
    t = agg + g2_ref[...] * y + b_ref[...]
    if act:
      t = jnp.where(t > 0, t, jnp.exp(jnp.minimum(t, 0.0)) - 1.0)
    z = jnp.dot(t, w_ref[...], preferred_element_type=jnp.float32)
    o0_ref[...] = z[:, :dhq]
    o1_ref[...] = z[:, dhq:]

  return pl.pallas_call(
      body,
      grid=(n // B,),
      in_specs=[pl.BlockSpec((B, dhp), lambda i: (i, 0)),
                pl.BlockSpec((B, dhp), lambda i: (i, 0)),
                pl.BlockSpec((B, dhp), lambda i: (i, 0)),
                pl.BlockSpec((B, dhp), lambda i: (i, 0)),
                pl.BlockSpec((B, 1), lambda i: (i, 0)),
                pl.BlockSpec((1, dp), lambda i: (0, 0)),
                pl.BlockSpec((dp, dq), lambda i: (0, 0))],
      out_specs=[pl.BlockSpec((B, dhq), lambda i: (i, 0))] * 2,
      out_shape=[jax.ShapeDtypeStruct((n, dhq), jnp.float32)] * 2)


def _mm_final(n, dp):
  B = 1000
  dhp = dp // 2

  def body(a0_ref, a1_ref, y0_ref, y1_ref, g2_ref, b_ref, o_ref):
    agg = jnp.concatenate([a0_ref[...], a1_ref[...]], axis=1)
    y = jnp.concatenate([y0_ref[...], y1_ref[...]], axis=1)
    o_ref[...] = agg + g2_ref[...] * y + b_ref[...]

  return pl.pallas_call(
      body,
      grid=(n // B,),
      in_specs=[pl.BlockSpec((B, dhp), lambda i: (i, 0)),
                pl.BlockSpec((B, dhp), lambda i: (i, 0)),
                pl.BlockSpec((B, dhp), lambda i: (i, 0)),
                pl.BlockSpec((B, dhp), lambda i: (i, 0)),
                pl.BlockSpec((B, 1), lambda i: (i, 0)),
                pl.BlockSpec((1, dp), lambda i: (0, 0))],
      out_specs=pl.BlockSpec((B, dp), lambda i: (i, 0)),
      out_shape=jax.ShapeDtypeStruct((n, dp), jnp.float32))


def kernel(x, edge_index, edge_attr, W1, b1, W2, b2, W3, b3, W4, b4,
           W5, b5, W6, b6):
  n, f_in = x.shape
  e = edge_index.shape[1]

  align = NC * NS * CHUNK
  ep = (e + align - 1) // align * align
  pad = ep - e
  src = jnp.concatenate([edge_index[0], jnp.zeros((pad,), jnp.int32)])
  dst = jnp.concatenate([edge_index[1], jnp.zeros((pad,), jnp.int32)])
  ea = jnp.concatenate([edge_attr, jnp.zeros((pad,), jnp.float32)])
  src2 = src.reshape(-1, SUB)
  dst2 = dst.reshape(-1, SUB)

  Ws = [W1, W2, W3, W4, W5, W6]
  bs = [b1, b2, b3, b4, b5, b6]
  PD = [32, 32, 32, 64, 128, 128]

  Wp, bp = [], []
  prev = f_in
  for l in range(6):
    W, b = Ws[l], bs[l]
    wpad = jnp.zeros((prev, PD[l]), jnp.float32)
    wpad = wpad.at[:W.shape[0], :W.shape[1]].set(W)
    bpad = jnp.zeros((1, PD[l]), jnp.float32).at[0, :b.shape[0]].set(b)
    Wp.append(wpad)
    bp.append(bpad)
    prev = PD[l]

  z1 = jnp.zeros((n,), jnp.float32)
  zeros = {dw: jnp.zeros((n, dw), jnp.float32) for dw in {16, 32, 64}}

  deg0, deg1 = _deg_kernel(n, ep)(dst2, ea, z1)
  g2d, g22d = _prep_kernel(80, n // 80)(deg0.reshape(80, -1),
                                        deg1.reshape(80, -1))
  g = g2d.reshape(n)
  g2 = g22d.reshape(n, 1)
  nrm = _norm_kernel(n, ep)(src, dst, ea, g)

  y0, y1 = _mm_first(n, f_in, PD[0])(x, Wp[0])
  for l in range(6):
    dw = PD[l] // 2
    a0, a1 = _agg_kernel(n, ep, dw)(y0, y1, src2, dst2, nrm, zeros[dw])
    if l < 5:
      act = l in (0, 1, 2, 4)
      y0, y1 = _mm_mid(n, PD[l], PD[l + 1], act)(
          a0, a1, y0, y1, g2, bp[l], Wp[l + 1])
    else:
      out = _mm_final(n, PD[5])(a0, a1, y0, y1, g2, bp[5])

  return out.reshape(-1, 128)


# speedup vs baseline: 7.9993x; 1.4393x over previous
import functools

import jax
import jax.numpy as jnp
from jax import lax
from jax.experimental import pallas as pl
from jax.experimental.pallas import tpu as pltpu
from jax.experimental.pallas import tpu_sc as plsc

NC = 2
NS = 16
LANES = 16
SUB = 80
CHUNK = 16 * SUB

_mesh = plsc.VectorSubcoreMesh(
    core_axis_name="c", subcore_axis_name="s", num_cores=NC, num_subcores=NS)
_sc_params = pltpu.CompilerParams(needs_layout_passes=False, use_tc_tiling_on_sc=False)


def _row_split(n):
  rpt = (n // NS) // 8 * 8
  last = n - rpt * (NS - 1)
  assert rpt % 8 == 0 and last % 8 == 0
  return rpt, last


def _deg_kernel(n, ep):
  epc = ep // NC
  ept = epc // NS
  NSUB = CHUNK // SUB
  nch = ept // CHUNK
  assert ept % CHUNK == 0

  @functools.partial(
      pl.kernel, mesh=_mesh,
      out_type=[jax.ShapeDtypeStruct((n,), jnp.float32) for _ in range(NC)],
      compiler_params=_sc_params,
      scratch_types=[
          pltpu.VMEM_SHARED((n,), jnp.float32),
          pltpu.VMEM((NSUB, SUB), jnp.int32),
          pltpu.VMEM((CHUNK,), jnp.float32),
          pltpu.SemaphoreType.DMA,
      ])
  def deg_k(dst2_hbm, ea_hbm, z1_hbm, deg0_hbm, deg1_hbm, deg_sh, d_v, w_v,
            dsem):
    c = lax.axis_index("c")
    s = lax.axis_index("s")

    @pl.when(s == 0)
    def _():
      pltpu.sync_copy(z1_hbm, deg_sh)
    plsc.subcore_barrier()

    base0 = c * epc + s * ept

    def chunk(k, _):
      base = pl.multiple_of(base0 + k * CHUNK, CHUNK)
      pltpu.sync_copy(dst2_hbm.at[pl.ds(pl.multiple_of(base // SUB, 8), NSUB), :], d_v)
      pltpu.sync_copy(ea_hbm.at[pl.ds(base, CHUNK)], w_v)
      sc = [pltpu.async_copy(w_v.at[pl.ds(j * SUB, SUB)],
                             deg_sh.at[d_v.at[j]], dsem, add=True)
            for j in range(NSUB)]
      for dd in sc:
        dd.wait()
      return 0

    lax.fori_loop(0, nch, chunk, 0)
    plsc.subcore_barrier()

    @pl.when(s == 0)
    def _():
      @pl.when(c == 0)
      def _():
        pltpu.sync_copy(deg_sh, deg0_hbm)
      @pl.when(c == 1)
      def _():
        pltpu.sync_copy(deg_sh, deg1_hbm)

  return deg_k


def _norm_kernel(n, ep):
  nw = NC * NS
  ept = ep // nw
  C = 2048
  nch = ept // C
  steps = C // LANES
  assert ept % C == 0

  @functools.partial(
      pl.kernel, mesh=_mesh,
      out_type=jax.ShapeDtypeStruct((ep,), jnp.float32),
      compiler_params=_sc_params,
      scratch_types=[
          pltpu.VMEM((n,), jnp.float32),
          pltpu.VMEM((C,), jnp.int32),
          pltpu.VMEM((C,), jnp.int32),
          pltpu.VMEM((C,), jnp.float32),
          pltpu.VMEM((C,), jnp.float32),
      ])
  def norm_k(src_hbm, dst_hbm, ea_hbm, g_hbm, norm_hbm, g_v, s_v, d_v, w_v, o_v):
    c = lax.axis_index("c")
    s = lax.axis_index("s")
    wid = s * NC + c
    pltpu.sync_copy(g_hbm, g_v)
    base0 = wid * ept

    def chunk(k, _):
      base = pl.multiple_of(base0 + k * C, C)
      pltpu.sync_copy(src_hbm.at[pl.ds(base, C)], s_v)
      pltpu.sync_copy(dst_hbm.at[pl.ds(base, C)], d_v)
      pltpu.sync_copy(ea_hbm.at[pl.ds(base, C)], w_v)

      def step(i, _):
        off = i * LANES
        si = s_v[pl.ds(off, LANES)]
        di = d_v[pl.ds(off, LANES)]
        wv = w_v[pl.ds(off, LANES)]
        o_v[pl.ds(off, LANES)] = (
            plsc.load_gather(g_v, [si]) * wv * plsc.load_gather(g_v, [di]))
        return 0

      lax.fori_loop(0, steps, step, 0)
      pltpu.sync_copy(o_v, norm_hbm.at[pl.ds(base, C)])
      return 0

    lax.fori_loop(0, nch, chunk, 0)

  return norm_k


def _agg_kernel(n, ep, dw):
  ept = ep // NS
  NSUB = CHUNK // SUB
  nch = ept // CHUNK
  rpt, rlast = _row_split(n)
  assert ept % CHUNK == 0 and dw % LANES == 0

  @functools.partial(
      pl.kernel, mesh=_mesh,
      out_type=[jax.ShapeDtypeStruct((n, dw), jnp.float32) for _ in range(NC)],
      compiler_params=_sc_params,
      scratch_types=[
          pltpu.VMEM_SHARED((n, dw), jnp.float32),
          pltpu.VMEM((NSUB, SUB), jnp.int32),
          pltpu.VMEM((NSUB, SUB), jnp.int32),
          pltpu.VMEM((CHUNK,), jnp.float32),
          pltpu.VMEM((CHUNK, dw), jnp.float32),
          pltpu.SemaphoreType.DMA,
      ])
  def agg_k(y0_hbm, y1_hbm, src2_hbm, dst2_hbm, nrm_hbm, z_hbm,
            a0_hbm, a1_hbm, acc_sh, s_v, d_v, n_v, rows_v, sem):
    c = lax.axis_index("c")
    s = lax.axis_index("s")

    @pl.when(s == 0)
    def _():
      pltpu.sync_copy(z_hbm, acc_sh)
    plsc.subcore_barrier()

    base0 = s * ept

    def chunk(k, _):
      base = pl.multiple_of(base0 + k * CHUNK, CHUNK)
      brow = pl.multiple_of(base // SUB, 8)
      st = [pltpu.async_copy(src2_hbm.at[pl.ds(brow, NSUB), :], s_v, sem),
            pltpu.async_copy(dst2_hbm.at[pl.ds(brow, NSUB), :], d_v, sem),
            pltpu.async_copy(nrm_hbm.at[pl.ds(base, CHUNK)], n_v, sem)]
      for dd in st:
        dd.wait()

      @pl.when(c == 0)
      def _():
        gd = [pltpu.async_copy(y0_hbm.at[s_v.at[j]],
                               rows_v.at[pl.ds(j * SUB, SUB), :], sem)
              for j in range(NSUB)]
        for dd in gd:
          dd.wait()
      @pl.when(c == 1)
      def _():
        gd = [pltpu.async_copy(y1_hbm.at[s_v.at[j]],
                               rows_v.at[pl.ds(j * SUB, SUB), :], sem)
              for j in range(NSUB)]
        for dd in gd:
          dd.wait()

      def srow(r, _):
        spl = plsc.load_gather(n_v, [jnp.full((LANES,), 0, jnp.int32) + r])
        for jj in range(dw // LANES):
          sl = pl.ds(jj * LANES, LANES)
          rows_v[r, sl] = rows_v[r, sl] * spl
        return 0

      lax.fori_loop(0, CHUNK, srow, 0)

      sc = [pltpu.async_copy(rows_v.at[pl.ds(j * SUB, SUB), :],
                             acc_sh.at[d_v.at[j]], sem, add=True)
            for j in range(NSUB)]
      for dd in sc:
        dd.wait()
      return 0

    lax.fori_loop(0, nch, chunk, 0)
    plsc.subcore_barrier()

    r0 = pl.multiple_of(s * rpt, 8)
    for cc, a_hbm in ((0, a0_hbm), (1, a1_hbm)):
      @pl.when((c == cc) & (s < NS - 1))
      def _():
        pltpu.sync_copy(acc_sh.at[pl.ds(r0, rpt), :],
                        a_hbm.at[pl.ds(r0, rpt), :])
      @pl.when((c == cc) & (s == NS - 1))
      def _():
        pltpu.sync_copy(acc_sh.at[pl.ds(r0, rlast), :],
                        a_hbm.at[pl.ds(r0, rlast), :])

  return agg_k


def _prep_kernel(r, q):
  def body(d0_ref, d1_ref, g_ref, g2_ref):
    deg = d0_ref[...] + d1_ref[...] + 1.0
    g = jnp.where(deg > 0, lax.rsqrt(deg), 0.0)
    g_ref[...] = g
    g2_ref[...] = g * g

  return pl.pallas_call(
      body,
      out_shape=[jax.ShapeDtypeStruct((r, q), jnp.float32)] * 2)


def _mm_first(n, din, dq):
  B = 1000
  dh = dq // 2

  def body(x_ref, w_ref, y0_ref, y1_ref):
    y = jnp.dot(x_ref[...], w_ref[...], preferred_element_type=jnp.float32)
    y0_ref[...] = y[:, :dh]
    y1_ref[...] = y[:, dh:]

  return pl.pallas_call(
      body,
      grid=(n // B,),
      in_specs=[pl.BlockSpec((B, din), lambda i: (i, 0)),
                pl.BlockSpec((din, dq), lambda i: (0, 0))],
      out_specs=[pl.BlockSpec((B, dh), lambda i: (i, 0))] * 2,
      out_shape=[jax.ShapeDtypeStruct((n, dh), jnp.float32)] * 2)


def _mm_mid(n, dp, dq, act):
  B = 1000
  dhp = dp // 2
  dhq = dq // 2

  def body(a0_ref, a1_ref, y0_ref, y1_ref, g2_ref, b_ref, w_ref,
           o0_ref, o1_ref):
    agg = jnp.concatenate([a0_ref[...], a1_ref[...]], axis=1)
    y = jnp.concatenate([y0_ref[...], y1_ref[...]], axis=1)
    t = agg + g2_ref[...] * y + b_ref[...]
    if act:
      t = jnp.where(t > 0, t, jnp.exp(jnp.minimum(t, 0.0)) - 1.0)
    z = jnp.dot(t, w_ref[...], preferred_element_type=jnp.float32)
    o0_ref[...] = z[:, :dhq]
    o1_ref[...] = z[:, dhq:]

  return pl.pallas_call(
      body,
      grid=(n // B,),
      in_specs=[pl.BlockSpec((B, dhp), lambda i: (i, 0)),
                pl.BlockSpec((B, dhp), lambda i: (i, 0)),
                pl.BlockSpec((B, dhp), lambda i: (i, 0)),
                pl.BlockSpec((B, dhp), lambda i: (i, 0)),
                pl.BlockSpec((B, 1), lambda i: (i, 0)),
                pl.BlockSpec((1, dp), lambda i: (0, 0)),
                pl.BlockSpec((dp, dq), lambda i: (0, 0))],
      out_specs=[pl.BlockSpec((B, dhq), lambda i: (i, 0))] * 2,
      out_shape=[jax.ShapeDtypeStruct((n, dhq), jnp.float32)] * 2)


def _mm_final(n, dp):
  B = 1000
  dhp = dp // 2

  def body(a0_ref, a1_ref, y0_ref, y1_ref, g2_ref, b_ref, o_ref):
    agg = jnp.concatenate([a0_ref[...], a1_ref[...]], axis=1)
    y = jnp.concatenate([y0_ref[...], y1_ref[...]], axis=1)
    o_ref[...] = agg + g2_ref[...] * y + b_ref[...]

  return pl.pallas_call(
      body,
      grid=(n // B,),
      in_specs=[pl.BlockSpec((B, dhp), lambda i: (i, 0)),
                pl.BlockSpec((B, dhp), lambda i: (i, 0)),
                pl.BlockSpec((B, dhp), lambda i: (i, 0)),
                pl.BlockSpec((B, dhp), lambda i: (i, 0)),
                pl.BlockSpec((B, 1), lambda i: (i, 0)),
                pl.BlockSpec((1, dp), lambda i: (0, 0))],
      out_specs=pl.BlockSpec((B, dp), lambda i: (i, 0)),
      out_shape=jax.ShapeDtypeStruct((n, dp), jnp.float32))


def kernel(x, edge_index, edge_attr, W1, b1, W2, b2, W3, b3, W4, b4,
           W5, b5, W6, b6):
  n, f_in = x.shape
  e = edge_index.shape[1]

  align = NC * NS * CHUNK
  ep = (e + align - 1) // align * align
  pad = ep - e
  src = jnp.concatenate([edge_index[0], jnp.zeros((pad,), jnp.int32)])
  dst = jnp.concatenate([edge_index[1], jnp.zeros((pad,), jnp.int32)])
  ea = jnp.concatenate([edge_attr, jnp.zeros((pad,), jnp.float32)])
  src2 = src.reshape(-1, SUB)
  dst2 = dst.reshape(-1, SUB)

  Ws = [W1, W2, W3, W4, W5, W6]
  bs = [b1, b2, b3, b4, b5, b6]
  PD = [32, 32, 32, 64, 128, 128]

  Wp, bp = [], []
  prev = f_in
  for l in range(6):
    W, b = Ws[l], bs[l]
    wpad = jnp.zeros((prev, PD[l]), jnp.float32)
    wpad = wpad.at[:W.shape[0], :W.shape[1]].set(W)
    bpad = jnp.zeros((1, PD[l]), jnp.float32).at[0, :b.shape[0]].set(b)
    Wp.append(wpad)
    bp.append(bpad)
    prev = PD[l]

  z1 = jnp.zeros((n,), jnp.float32)
  zeros = {dw: jnp.zeros((n, dw), jnp.float32) for dw in {16, 32, 64}}

  deg0, deg1 = _deg_kernel(n, ep)(dst2, ea, z1)
  g2d, g22d = _prep_kernel(80, n // 80)(deg0.reshape(80, -1),
                                        deg1.reshape(80, -1))
  g = g2d.reshape(n)
  g2 = g22d.reshape(n, 1)
  nrm = _norm_kernel(n, ep)(src, dst, ea, g)

  y0, y1 = _mm_first(n, f_in, PD[0])(x, Wp[0])
  for l in range(6):
    dw = PD[l] // 2
    a0, a1 = _agg_kernel(n, ep, dw)(y0, y1, src2, dst2, nrm, zeros[dw])
    if l < 5:
      act = l in (0, 1, 2, 4)
      y0, y1 = _mm_mid(n, PD[l], PD[l + 1], act)(
          a0, a1, y0, y1, g2, bp[l], Wp[l + 1])
    else:
      out = _mm_final(n, PD[5])(a0, a1, y0, y1, g2, bp[5])

  return out.reshape(-1, 128)


# speedup vs baseline: 11.1690x; 1.3963x over previous
import functools

import jax
import jax.numpy as jnp
from jax import lax
from jax.experimental import pallas as pl
from jax.experimental.pallas import tpu as pltpu
from jax.experimental.pallas import tpu_sc as plsc

NC = 2
NS = 16
LANES = 16
SUB = 80
CHUNK = 16 * SUB

_mesh = plsc.VectorSubcoreMesh(
    core_axis_name="c", subcore_axis_name="s", num_cores=NC, num_subcores=NS)
_sc_params = pltpu.CompilerParams(needs_layout_passes=False, use_tc_tiling_on_sc=False)


def _row_split(n):
  rpt = (n // NS) // 8 * 8
  last = n - rpt * (NS - 1)
  assert rpt % 8 == 0 and last % 8 == 0
  return rpt, last


def _deg_kernel(n, ep):
  epc = ep // NC
  ept = epc // NS
  NSUB = CHUNK // SUB
  nch = ept // CHUNK
  assert ept % CHUNK == 0

  @functools.partial(
      pl.kernel, mesh=_mesh,
      out_type=[jax.ShapeDtypeStruct((n,), jnp.float32) for _ in range(NC)],
      compiler_params=_sc_params,
      scratch_types=[
          pltpu.VMEM_SHARED((n,), jnp.float32),
          pltpu.VMEM((NSUB, SUB), jnp.int32),
          pltpu.VMEM((CHUNK,), jnp.float32),
          pltpu.SemaphoreType.DMA,
      ])
  def deg_k(dst2_hbm, ea_hbm, z1_hbm, deg0_hbm, deg1_hbm, deg_sh, d_v, w_v,
            dsem):
    c = lax.axis_index("c")
    s = lax.axis_index("s")

    @pl.when(s == 0)
    def _():
      pltpu.sync_copy(z1_hbm, deg_sh)
    plsc.subcore_barrier()

    base0 = c * epc + s * ept

    def chunk(k, _):
      base = pl.multiple_of(base0 + k * CHUNK, CHUNK)
      pltpu.sync_copy(dst2_hbm.at[pl.ds(pl.multiple_of(base // SUB, 8), NSUB), :], d_v)
      pltpu.sync_copy(ea_hbm.at[pl.ds(base, CHUNK)], w_v)
      sc = [pltpu.async_copy(w_v.at[pl.ds(j * SUB, SUB)],
                             deg_sh.at[d_v.at[j]], dsem, add=True)
            for j in range(NSUB)]
      for dd in sc:
        dd.wait()
      return 0

    lax.fori_loop(0, nch, chunk, 0)
    plsc.subcore_barrier()

    @pl.when(s == 0)
    def _():
      @pl.when(c == 0)
      def _():
        pltpu.sync_copy(deg_sh, deg0_hbm)
      @pl.when(c == 1)
      def _():
        pltpu.sync_copy(deg_sh, deg1_hbm)

  return deg_k


def _norm_kernel(n, ep):
  nw = NC * NS
  ept = ep // nw
  C = 2048
  nch = ept // C
  steps = C // LANES
  assert ept % C == 0

  @functools.partial(
      pl.kernel, mesh=_mesh,
      out_type=jax.ShapeDtypeStruct((ep,), jnp.float32),
      compiler_params=_sc_params,
      scratch_types=[
          pltpu.VMEM((n,), jnp.float32),
          pltpu.VMEM((C,), jnp.int32),
          pltpu.VMEM((C,), jnp.int32),
          pltpu.VMEM((C,), jnp.float32),
          pltpu.VMEM((C,), jnp.float32),
      ])
  def norm_k(src_hbm, dst_hbm, ea_hbm, g_hbm, norm_hbm, g_v, s_v, d_v, w_v, o_v):
    c = lax.axis_index("c")
    s = lax.axis_index("s")
    wid = s * NC + c
    pltpu.sync_copy(g_hbm, g_v)
    base0 = wid * ept

    def chunk(k, _):
      base = pl.multiple_of(base0 + k * C, C)
      pltpu.sync_copy(src_hbm.at[pl.ds(base, C)], s_v)
      pltpu.sync_copy(dst_hbm.at[pl.ds(base, C)], d_v)
      pltpu.sync_copy(ea_hbm.at[pl.ds(base, C)], w_v)

      def step(i, _):
        off = i * LANES
        si = s_v[pl.ds(off, LANES)]
        di = d_v[pl.ds(off, LANES)]
        wv = w_v[pl.ds(off, LANES)]
        o_v[pl.ds(off, LANES)] = (
            plsc.load_gather(g_v, [si]) * wv * plsc.load_gather(g_v, [di]))
        return 0

      lax.fori_loop(0, steps, step, 0)
      pltpu.sync_copy(o_v, norm_hbm.at[pl.ds(base, C)])
      return 0

    lax.fori_loop(0, nch, chunk, 0)

  return norm_k


def _agg_kernel(n, ep, dw):
  ept = ep // NS
  CH = 20480 // dw
  SUBA = 40 if CH // 80 < 8 else 80
  NSUB = CH // SUBA
  nch = ept // CH
  rpt, rlast = _row_split(n)
  assert ept % CH == 0 and nch % 4 == 0 and dw % LANES == 0

  @functools.partial(
      pl.kernel, mesh=_mesh,
      out_type=[jax.ShapeDtypeStruct((n, dw), jnp.float32) for _ in range(NC)],
      compiler_params=_sc_params,
      scratch_types=[
          pltpu.VMEM_SHARED((n, dw), jnp.float32),
          pltpu.VMEM((4, NSUB, SUBA), jnp.int32),
          pltpu.VMEM((4, NSUB, SUBA), jnp.int32),
          pltpu.VMEM((4, CH), jnp.float32),
          pltpu.VMEM((4, CH, dw), jnp.float32),
          pltpu.SemaphoreType.DMA,
          pltpu.SemaphoreType.DMA,
          pltpu.SemaphoreType.DMA,
      ])
  def agg_k(y0_hbm, y1_hbm, src2_hbm, dst2_hbm, nrm_hbm, z_hbm,
            a0_hbm, a1_hbm, acc_sh, s_v, d_v, n_v, rows_v,
            sem_st, sem_g, sem_s):
    c = lax.axis_index("c")
    s = lax.axis_index("s")

    @pl.when(s == 0)
    def _():
      pltpu.sync_copy(z_hbm, acc_sh)
    plsc.subcore_barrier()

    base0 = s * ept

    def base_of(k):
      return pl.multiple_of(base0 + k * CH, CH)

    def stage_copies(k, u, fire):
      base = base_of(k)
      brow = pl.multiple_of(base // SUBA, NSUB)
      f = pltpu.async_copy if fire else pltpu.make_async_copy
      return [f(src2_hbm.at[pl.ds(brow, NSUB), :], s_v.at[u], sem_st),
              f(dst2_hbm.at[pl.ds(brow, NSUB), :], d_v.at[u], sem_st),
              f(nrm_hbm.at[pl.ds(base, CH)], n_v.at[u], sem_st)]

    def issue_stage(k, u):
      stage_copies(k, u, True)

    def wait_stage(k, u):
      for dd in stage_copies(k, u, False):
        dd.wait()

    def gather_copies(y_hbm, u, fire):
      f = pltpu.async_copy if fire else pltpu.make_async_copy
      return [f(y_hbm.at[s_v.at[u, j]],
                rows_v.at[u, pl.ds(j * SUBA, SUBA), :], sem_g)
              for j in range(NSUB)]

    def issue_gather(u):
      @pl.when(c == 0)
      def _():
        gather_copies(y0_hbm, u, True)
      @pl.when(c == 1)
      def _():
        gather_copies(y1_hbm, u, True)

    def wait_gather(u):
      for dd in gather_copies(y0_hbm, u, False):
        dd.wait()

    def scatter_copies(u, fire):
      if fire:
        return [pltpu.async_copy(rows_v.at[u, pl.ds(j * SUBA, SUBA), :],
                                 acc_sh.at[d_v.at[u, j]], sem_s, add=True)
                for j in range(NSUB)]
      return [pltpu.make_async_copy(rows_v.at[u, pl.ds(j * SUBA, SUBA), :],
                                    acc_sh.at[d_v.at[u, j]], sem_s)
              for j in range(NSUB)]

    def issue_scatter(u):
      scatter_copies(u, True)

    def wait_scatter(u):
      for dd in scatter_copies(u, False):
        dd.wait()

    def scale(u):
      def srow(r, _):
        spl = plsc.load_gather(n_v.at[u], [jnp.full((LANES,), 0, jnp.int32) + r])
        for jj in range(dw // LANES):
          sl = pl.ds(jj * LANES, LANES)
          rows_v[u, r, sl] = rows_v[u, r, sl] * spl
        return 0
      lax.fori_loop(0, CH, srow, 0)

    issue_stage(0, 0)
    issue_stage(1, 1)
    wait_stage(0, 0)
    issue_gather(0)

    def body(m, _):
      for u in range(4):
        k = 4 * m + u
        up1 = (u + 1) % 4
        up2 = (u + 2) % 4

        @pl.when(k >= 2)
        def _():
          wait_scatter(up2)
        @pl.when(k + 2 < nch)
        def _():
          issue_stage(k + 2, up2)
        @pl.when(k + 1 < nch)
        def _():
          wait_stage(k + 1, up1)
          issue_gather(up1)

        wait_gather(u)
        scale(u)
        issue_scatter(u)
      return 0

    lax.fori_loop(0, nch // 4, body, 0)
    wait_scatter((nch - 2) % 4)
    wait_scatter((nch - 1) % 4)
    plsc.subcore_barrier()

    r0 = pl.multiple_of(s * rpt, 8)
    for cc, a_hbm in ((0, a0_hbm), (1, a1_hbm)):
      @pl.when((c == cc) & (s < NS - 1))
      def _():
        pltpu.sync_copy(acc_sh.at[pl.ds(r0, rpt), :],
                        a_hbm.at[pl.ds(r0, rpt), :])
      @pl.when((c == cc) & (s == NS - 1))
      def _():
        pltpu.sync_copy(acc_sh.at[pl.ds(r0, rlast), :],
                        a_hbm.at[pl.ds(r0, rlast), :])

  return agg_k


def _prep_kernel(r, q):
  def body(d0_ref, d1_ref, g_ref, g2_ref):
    deg = d0_ref[...] + d1_ref[...] + 1.0
    g = jnp.where(deg > 0, lax.rsqrt(deg), 0.0)
    g_ref[...] = g
    g2_ref[...] = g * g

  return pl.pallas_call(
      body,
      out_shape=[jax.ShapeDtypeStruct((r, q), jnp.float32)] * 2)


def _mm_first(n, din, dq):
  B = 1000
  dh = dq // 2

  def body(x_ref, w_ref, y0_ref, y1_ref):
    y = jnp.dot(x_ref[...], w_ref[...], preferred_element_type=jnp.float32)
    y0_ref[...] = y[:, :dh]
    y1_ref[...] = y[:, dh:]

  return pl.pallas_call(
      body,
      grid=(n // B,),
      in_specs=[pl.BlockSpec((B, din), lambda i: (i, 0)),
                pl.BlockSpec((din, dq), lambda i: (0, 0))],
      out_specs=[pl.BlockSpec((B, dh), lambda i: (i, 0))] * 2,
      out_shape=[jax.ShapeDtypeStruct((n, dh), jnp.float32)] * 2)


def _mm_mid(n, dp, dq, act):
  B = 1000
  dhp = dp // 2
  dhq = dq // 2

  def body(a0_ref, a1_ref, y0_ref, y1_ref, g2_ref, b_ref, w_ref,
           o0_ref, o1_ref):
    agg = jnp.concatenate([a0_ref[...], a1_ref[...]], axis=1)
    y = jnp.concatenate([y0_ref[...], y1_ref[...]], axis=1)
    t = agg + g2_ref[...] * y + b_ref[...]
    if act:
      t = jnp.where(t > 0, t, jnp.exp(jnp.minimum(t, 0.0)) - 1.0)
    z = jnp.dot(t, w_ref[...], preferred_element_type=jnp.float32)
    o0_ref[...] = z[:, :dhq]
    o1_ref[...] = z[:, dhq:]

  return pl.pallas_call(
      body,
      grid=(n // B,),
      in_specs=[pl.BlockSpec((B, dhp), lambda i: (i, 0)),
                pl.BlockSpec((B, dhp), lambda i: (i, 0)),
                pl.BlockSpec((B, dhp), lambda i: (i, 0)),
                pl.BlockSpec((B, dhp), lambda i: (i, 0)),
                pl.BlockSpec((B, 1), lambda i: (i, 0)),
                pl.BlockSpec((1, dp), lambda i: (0, 0)),
                pl.BlockSpec((dp, dq), lambda i: (0, 0))],
      out_specs=[pl.BlockSpec((B, dhq), lambda i: (i, 0))] * 2,
      out_shape=[jax.ShapeDtypeStruct((n, dhq), jnp.float32)] * 2)


def _mm_final(n, dp):
  B = 1000
  dhp = dp // 2

  def body(a0_ref, a1_ref, y0_ref, y1_ref, g2_ref, b_ref, o_ref):
    agg = jnp.concatenate([a0_ref[...], a1_ref[...]], axis=1)
    y = jnp.concatenate([y0_ref[...], y1_ref[...]], axis=1)
    o_ref[...] = agg + g2_ref[...] * y + b_ref[...]

  return pl.pallas_call(
      body,
      grid=(n // B,),
      in_specs=[pl.BlockSpec((B, dhp), lambda i: (i, 0)),
                pl.BlockSpec((B, dhp), lambda i: (i, 0)),
                pl.BlockSpec((B, dhp), lambda i: (i, 0)),
                pl.BlockSpec((B, dhp), lambda i: (i, 0)),
                pl.BlockSpec((B, 1), lambda i: (i, 0)),
                pl.BlockSpec((1, dp), lambda i: (0, 0))],
      out_specs=pl.BlockSpec((B, dp), lambda i: (i, 0)),
      out_shape=jax.ShapeDtypeStruct((n, dp), jnp.float32))


def kernel(x, edge_index, edge_attr, W1, b1, W2, b2, W3, b3, W4, b4,
           W5, b5, W6, b6):
  n, f_in = x.shape
  e = edge_index.shape[1]

  align = NC * NS * CHUNK
  ep = (e + align - 1) // align * align
  pad = ep - e
  src = jnp.concatenate([edge_index[0], jnp.zeros((pad,), jnp.int32)])
  dst = jnp.concatenate([edge_index[1], jnp.zeros((pad,), jnp.int32)])
  ea = jnp.concatenate([edge_attr, jnp.zeros((pad,), jnp.float32)])
  iviews = {40: (src.reshape(-1, 40), dst.reshape(-1, 40)),
            80: (src.reshape(-1, 80), dst.reshape(-1, 80))}
  src2, dst2 = iviews[80]

  Ws = [W1, W2, W3, W4, W5, W6]
  bs = [b1, b2, b3, b4, b5, b6]
  PD = [32, 32, 32, 64, 128, 128]

  Wp, bp = [], []
  prev = f_in
  for l in range(6):
    W, b = Ws[l], bs[l]
    wpad = jnp.zeros((prev, PD[l]), jnp.float32)
    wpad = wpad.at[:W.shape[0], :W.shape[1]].set(W)
    bpad = jnp.zeros((1, PD[l]), jnp.float32).at[0, :b.shape[0]].set(b)
    Wp.append(wpad)
    bp.append(bpad)
    prev = PD[l]

  z1 = jnp.zeros((n,), jnp.float32)
  zeros = {dw: jnp.zeros((n, dw), jnp.float32) for dw in {16, 32, 64}}

  deg0, deg1 = _deg_kernel(n, ep)(dst2, ea, z1)
  g2d, g22d = _prep_kernel(80, n // 80)(deg0.reshape(80, -1),
                                        deg1.reshape(80, -1))
  g = g2d.reshape(n)
  g2 = g22d.reshape(n, 1)
  nrm = _norm_kernel(n, ep)(src, dst, ea, g)

  y0, y1 = _mm_first(n, f_in, PD[0])(x, Wp[0])
  for l in range(6):
    dw = PD[l] // 2
    suba = 40 if (20480 // dw) // 80 < 8 else 80
    srcv, dstv = iviews[suba]
    a0, a1 = _agg_kernel(n, ep, dw)(y0, y1, srcv, dstv, nrm, zeros[dw])
    if l < 5:
      act = l in (0, 1, 2, 4)
      y0, y1 = _mm_mid(n, PD[l], PD[l + 1], act)(
          a0, a1, y0, y1, g2, bp[l], Wp[l + 1])
    else:
      out = _mm_final(n, PD[5])(a0, a1, y0, y1, g2, bp[5])

  return out.reshape(-1, 128)


# speedup vs baseline: 14.0154x; 1.2548x over previous
import functools

import jax
import jax.numpy as jnp
from jax import lax
from jax.experimental import pallas as pl
from jax.experimental.pallas import tpu as pltpu
from jax.experimental.pallas import tpu_sc as plsc

NC = 2
NS = 16
LANES = 16
SUB = 80
CHUNK = 16 * SUB

_mesh = plsc.VectorSubcoreMesh(
    core_axis_name="c", subcore_axis_name="s", num_cores=NC, num_subcores=NS)
_sc_params = pltpu.CompilerParams(needs_layout_passes=False, use_tc_tiling_on_sc=False)


def _row_split(n):
  rpt = (n // NS) // 8 * 8
  last = n - rpt * (NS - 1)
  assert rpt % 8 == 0 and last % 8 == 0
  return rpt, last


def _deg_kernel(n, ep):
  epc = ep // NC
  ept = epc // NS
  NSUB = CHUNK // SUB
  nch = ept // CHUNK
  assert ept % CHUNK == 0

  @functools.partial(
      pl.kernel, mesh=_mesh,
      out_type=[jax.ShapeDtypeStruct((n,), jnp.float32) for _ in range(NC)],
      compiler_params=_sc_params,
      scratch_types=[
          pltpu.VMEM_SHARED((n,), jnp.float32),
          pltpu.VMEM((NSUB, SUB), jnp.int32),
          pltpu.VMEM((CHUNK,), jnp.float32),
          pltpu.SemaphoreType.DMA,
      ])
  def deg_k(dst2_hbm, ea_hbm, z1_hbm, deg0_hbm, deg1_hbm, deg_sh, d_v, w_v,
            dsem):
    c = lax.axis_index("c")
    s = lax.axis_index("s")

    @pl.when(s == 0)
    def _():
      pltpu.sync_copy(z1_hbm, deg_sh)
    plsc.subcore_barrier()

    base0 = c * epc + s * ept

    def chunk(k, _):
      base = pl.multiple_of(base0 + k * CHUNK, CHUNK)
      pltpu.sync_copy(dst2_hbm.at[pl.ds(pl.multiple_of(base // SUB, 8), NSUB), :], d_v)
      pltpu.sync_copy(ea_hbm.at[pl.ds(base, CHUNK)], w_v)
      sc = [pltpu.async_copy(w_v.at[pl.ds(j * SUB, SUB)],
                             deg_sh.at[d_v.at[j]], dsem, add=True)
            for j in range(NSUB)]
      for dd in sc:
        dd.wait()
      return 0

    lax.fori_loop(0, nch, chunk, 0)
    plsc.subcore_barrier()

    @pl.when(s == 0)
    def _():
      @pl.when(c == 0)
      def _():
        pltpu.sync_copy(deg_sh, deg0_hbm)
      @pl.when(c == 1)
      def _():
        pltpu.sync_copy(deg_sh, deg1_hbm)

  return deg_k


def _norm_kernel(n, ep):
  nw = NC * NS
  ept = ep // nw
  C = 2048
  nch = ept // C
  steps = C // LANES
  assert ept % C == 0

  @functools.partial(
      pl.kernel, mesh=_mesh,
      out_type=jax.ShapeDtypeStruct((ep,), jnp.float32),
      compiler_params=_sc_params,
      scratch_types=[
          pltpu.VMEM((n,), jnp.float32),
          pltpu.VMEM((C,), jnp.int32),
          pltpu.VMEM((C,), jnp.int32),
          pltpu.VMEM((C,), jnp.float32),
          pltpu.VMEM((C,), jnp.float32),
      ])
  def norm_k(src_hbm, dst_hbm, ea_hbm, g_hbm, norm_hbm, g_v, s_v, d_v, w_v, o_v):
    c = lax.axis_index("c")
    s = lax.axis_index("s")
    wid = s * NC + c
    pltpu.sync_copy(g_hbm, g_v)
    base0 = wid * ept

    def chunk(k, _):
      base = pl.multiple_of(base0 + k * C, C)
      pltpu.sync_copy(src_hbm.at[pl.ds(base, C)], s_v)
      pltpu.sync_copy(dst_hbm.at[pl.ds(base, C)], d_v)
      pltpu.sync_copy(ea_hbm.at[pl.ds(base, C)], w_v)

      def step(i, _):
        off = i * LANES
        si = s_v[pl.ds(off, LANES)]
        di = d_v[pl.ds(off, LANES)]
        wv = w_v[pl.ds(off, LANES)]
        o_v[pl.ds(off, LANES)] = (
            plsc.load_gather(g_v, [si]) * wv * plsc.load_gather(g_v, [di]))
        return 0

      lax.fori_loop(0, steps, step, 0)
      pltpu.sync_copy(o_v, norm_hbm.at[pl.ds(base, C)])
      return 0

    lax.fori_loop(0, nch, chunk, 0)

  return norm_k


def _agg_kernel(n, ep, dw):
  ept = ep // NS
  CH = 20480 // dw
  SUBA = 40 if CH // 80 < 8 else 80
  NSUB = CH // SUBA
  nch = ept // CH
  rpt, rlast = _row_split(n)
  assert ept % CH == 0 and nch % 4 == 0 and dw % LANES == 0

  @functools.partial(
      pl.kernel, mesh=_mesh,
      out_type=[jax.ShapeDtypeStruct((n, dw), jnp.float32) for _ in range(NC)],
      compiler_params=_sc_params,
      scratch_types=[
          pltpu.VMEM_SHARED((n, dw), jnp.float32),
          pltpu.VMEM((4, NSUB, SUBA), jnp.int32),
          pltpu.VMEM((4, NSUB, SUBA), jnp.int32),
          pltpu.VMEM((4, CH), jnp.float32),
          pltpu.VMEM((4, CH, dw), jnp.float32),
          pltpu.SemaphoreType.DMA,
          pltpu.SemaphoreType.DMA,
          pltpu.SemaphoreType.DMA,
      ])
  def agg_k(y0_hbm, y1_hbm, src2_hbm, dst2_hbm, nrm_hbm, z_hbm,
            a0_hbm, a1_hbm, acc_sh, s_v, d_v, n_v, rows_v,
            sem_st, sem_g, sem_s):
    c = lax.axis_index("c")
    s = lax.axis_index("s")

    @pl.when(s == 0)
    def _():
      pltpu.sync_copy(z_hbm, acc_sh)
    plsc.subcore_barrier()

    base0 = s * ept

    def base_of(k):
      return pl.multiple_of(base0 + k * CH, CH)

    def stage_copies(k, u, fire):
      base = base_of(k)
      brow = pl.multiple_of(base // SUBA, NSUB)
      f = pltpu.async_copy if fire else pltpu.make_async_copy
      return [f(src2_hbm.at[pl.ds(brow, NSUB), :], s_v.at[u], sem_st),
              f(dst2_hbm.at[pl.ds(brow, NSUB), :], d_v.at[u], sem_st),
              f(nrm_hbm.at[pl.ds(base, CH)], n_v.at[u], sem_st)]

    def issue_stage(k, u):
      stage_copies(k, u, True)

    def wait_stage(k, u):
      for dd in stage_copies(k, u, False):
        dd.wait()

    def gather_copies(y_hbm, u, fire):
      f = pltpu.async_copy if fire else pltpu.make_async_copy
      return [f(y_hbm.at[s_v.at[u, j]],
                rows_v.at[u, pl.ds(j * SUBA, SUBA), :], sem_g)
              for j in range(NSUB)]

    def issue_gather(u):
      @pl.when(c == 0)
      def _():
        gather_copies(y0_hbm, u, True)
      @pl.when(c == 1)
      def _():
        gather_copies(y1_hbm, u, True)

    def wait_gather(u):
      for dd in gather_copies(y0_hbm, u, False):
        dd.wait()

    def scatter_copies(u, fire):
      if fire:
        return [pltpu.async_copy(rows_v.at[u, pl.ds(j * SUBA, SUBA), :],
                                 acc_sh.at[d_v.at[u, j]], sem_s, add=True)
                for j in range(NSUB)]
      return [pltpu.make_async_copy(rows_v.at[u, pl.ds(j * SUBA, SUBA), :],
                                    acc_sh.at[d_v.at[u, j]], sem_s)
              for j in range(NSUB)]

    def issue_scatter(u):
      scatter_copies(u, True)

    def wait_scatter(u):
      for dd in scatter_copies(u, False):
        dd.wait()

    RU = 8

    def scale(u):
      def srow(i, _):
        r0 = pl.multiple_of(i * RU, RU)
        spls = [plsc.load_gather(n_v.at[u],
                                 [jnp.full((LANES,), t, jnp.int32) + r0])
                for t in range(RU)]
        for t in range(RU):
          for jj in range(dw // LANES):
            sl = pl.ds(jj * LANES, LANES)
            rows_v[u, r0 + t, sl] = rows_v[u, r0 + t, sl] * spls[t]
        return 0
      lax.fori_loop(0, CH // RU, srow, 0)

    issue_stage(0, 0)
    issue_stage(1, 1)
    wait_stage(0, 0)
    issue_gather(0)

    def body(m, _):
      for u in range(4):
        k = 4 * m + u
        up1 = (u + 1) % 4
        up2 = (u + 2) % 4

        @pl.when(k >= 2)
        def _():
          wait_scatter(up2)
        @pl.when(k + 2 < nch)
        def _():
          issue_stage(k + 2, up2)
        @pl.when(k + 1 < nch)
        def _():
          wait_stage(k + 1, up1)
          issue_gather(up1)

        wait_gather(u)
        scale(u)
        issue_scatter(u)
      return 0

    lax.fori_loop(0, nch // 4, body, 0)
    wait_scatter((nch - 2) % 4)
    wait_scatter((nch - 1) % 4)
    plsc.subcore_barrier()

    r0 = pl.multiple_of(s * rpt, 8)
    for cc, a_hbm in ((0, a0_hbm), (1, a1_hbm)):
      @pl.when((c == cc) & (s < NS - 1))
      def _():
        pltpu.sync_copy(acc_sh.at[pl.ds(r0, rpt), :],
                        a_hbm.at[pl.ds(r0, rpt), :])
      @pl.when((c == cc) & (s == NS - 1))
      def _():
        pltpu.sync_copy(acc_sh.at[pl.ds(r0, rlast), :],
                        a_hbm.at[pl.ds(r0, rlast), :])

  return agg_k


def _prep_kernel(r, q):
  def body(d0_ref, d1_ref, g_ref, g2_ref):
    deg = d0_ref[...] + d1_ref[...] + 1.0
    g = jnp.where(deg > 0, lax.rsqrt(deg), 0.0)
    g_ref[...] = g
    g2_ref[...] = g * g

  return pl.pallas_call(
      body,
      out_shape=[jax.ShapeDtypeStruct((r, q), jnp.float32)] * 2)


def _mm_first(n, din, dq):
  B = 1000
  dh = dq // 2

  def body(x_ref, w_ref, y0_ref, y1_ref):
    y = jnp.dot(x_ref[...], w_ref[...], preferred_element_type=jnp.float32)
    y0_ref[...] = y[:, :dh]
    y1_ref[...] = y[:, dh:]

  return pl.pallas_call(
      body,
      grid=(n // B,),
      in_specs=[pl.BlockSpec((B, din), lambda i: (i, 0)),
                pl.BlockSpec((din, dq), lambda i: (0, 0))],
      out_specs=[pl.BlockSpec((B, dh), lambda i: (i, 0))] * 2,
      out_shape=[jax.ShapeDtypeStruct((n, dh), jnp.float32)] * 2)


def _mm_mid(n, dp, dq, act):
  B = 1000
  dhp = dp // 2
  dhq = dq // 2

  def body(a0_ref, a1_ref, y0_ref, y1_ref, g2_ref, b_ref, w_ref,
           o0_ref, o1_ref):
    agg = jnp.concatenate([a0_ref[...], a1_ref[...]], axis=1)
    y = jnp.concatenate([y0_ref[...], y1_ref[...]], axis=1)
    t = agg + g2_ref[...] * y + b_ref[...]
    if act:
      t = jnp.where(t > 0, t, jnp.exp(jnp.minimum(t, 0.0)) - 1.0)
    z = jnp.dot(t, w_ref[...], preferred_element_type=jnp.float32)
    o0_ref[...] = z[:, :dhq]
    o1_ref[...] = z[:, dhq:]

  return pl.pallas_call(
      body,
      grid=(n // B,),
      in_specs=[pl.BlockSpec((B, dhp), lambda i: (i, 0)),
                pl.BlockSpec((B, dhp), lambda i: (i, 0)),
                pl.BlockSpec((B, dhp), lambda i: (i, 0)),
                pl.BlockSpec((B, dhp), lambda i: (i, 0)),
                pl.BlockSpec((B, 1), lambda i: (i, 0)),
                pl.BlockSpec((1, dp), lambda i: (0, 0)),
                pl.BlockSpec((dp, dq), lambda i: (0, 0))],
      out_specs=[pl.BlockSpec((B, dhq), lambda i: (i, 0))] * 2,
      out_shape=[jax.ShapeDtypeStruct((n, dhq), jnp.float32)] * 2)


def _mm_final(n, dp):
  B = 1000
  dhp = dp // 2

  def body(a0_ref, a1_ref, y0_ref, y1_ref, g2_ref, b_ref, o_ref):
    agg = jnp.concatenate([a0_ref[...], a1_ref[...]], axis=1)
    y = jnp.concatenate([y0_ref[...], y1_ref[...]], axis=1)
    o_ref[...] = agg + g2_ref[...] * y + b_ref[...]

  return pl.pallas_call(
      body,
      grid=(n // B,),
      in_specs=[pl.BlockSpec((B, dhp), lambda i: (i, 0)),
                pl.BlockSpec((B, dhp), lambda i: (i, 0)),
                pl.BlockSpec((B, dhp), lambda i: (i, 0)),
                pl.BlockSpec((B, dhp), lambda i: (i, 0)),
                pl.BlockSpec((B, 1), lambda i: (i, 0)),
                pl.BlockSpec((1, dp), lambda i: (0, 0))],
      out_specs=pl.BlockSpec((B, dp), lambda i: (i, 0)),
      out_shape=jax.ShapeDtypeStruct((n, dp), jnp.float32))


def kernel(x, edge_index, edge_attr, W1, b1, W2, b2, W3, b3, W4, b4,
           W5, b5, W6, b6):
  n, f_in = x.shape
  e = edge_index.shape[1]

  align = NC * NS * CHUNK
  ep = (e + align - 1) // align * align
  pad = ep - e
  src = jnp.concatenate([edge_index[0], jnp.zeros((pad,), jnp.int32)])
  dst = jnp.concatenate([edge_index[1], jnp.zeros((pad,), jnp.int32)])
  ea = jnp.concatenate([edge_attr, jnp.zeros((pad,), jnp.float32)])
  iviews = {40: (src.reshape(-1, 40), dst.reshape(-1, 40)),
            80: (src.reshape(-1, 80), dst.reshape(-1, 80))}
  src2, dst2 = iviews[80]

  Ws = [W1, W2, W3, W4, W5, W6]
  bs = [b1, b2, b3, b4, b5, b6]
  PD = [32, 32, 32, 64, 128, 128]

  Wp, bp = [], []
  prev = f_in
  for l in range(6):
    W, b = Ws[l], bs[l]
    wpad = jnp.zeros((prev, PD[l]), jnp.float32)
    wpad = wpad.at[:W.shape[0], :W.shape[1]].set(W)
    bpad = jnp.zeros((1, PD[l]), jnp.float32).at[0, :b.shape[0]].set(b)
    Wp.append(wpad)
    bp.append(bpad)
    prev = PD[l]

  z1 = jnp.zeros((n,), jnp.float32)
  zeros = {dw: jnp.zeros((n, dw), jnp.float32) for dw in {16, 32, 64}}

  deg0, deg1 = _deg_kernel(n, ep)(dst2, ea, z1)
  g2d, g22d = _prep_kernel(80, n // 80)(deg0.reshape(80, -1),
                                        deg1.reshape(80, -1))
  g = g2d.reshape(n)
  g2 = g22d.reshape(n, 1)
  nrm = _norm_kernel(n, ep)(src, dst, ea, g)

  y0, y1 = _mm_first(n, f_in, PD[0])(x, Wp[0])
  for l in range(6):
    dw = PD[l] // 2
    suba = 40 if (20480 // dw) // 80 < 8 else 80
    srcv, dstv = iviews[suba]
    a0, a1 = _agg_kernel(n, ep, dw)(y0, y1, srcv, dstv, nrm, zeros[dw])
    if l < 5:
      act = l in (0, 1, 2, 4)
      y0, y1 = _mm_mid(n, PD[l], PD[l + 1], act)(
          a0, a1, y0, y1, g2, bp[l], Wp[l + 1])
    else:
      out = _mm_final(n, PD[5])(a0, a1, y0, y1, g2, bp[5])

  return out.reshape(-1, 128)


# speedup vs baseline: 14.2061x; 1.0136x over previous
import functools

import jax
import jax.numpy as jnp
from jax import lax
from jax.experimental import pallas as pl
from jax.experimental.pallas import tpu as pltpu
from jax.experimental.pallas import tpu_sc as plsc

NC = 2
NS = 16
LANES = 16
SUB = 80
CHUNK = 16 * SUB

_mesh = plsc.VectorSubcoreMesh(
    core_axis_name="c", subcore_axis_name="s", num_cores=NC, num_subcores=NS)
_sc_params = pltpu.CompilerParams(needs_layout_passes=False, use_tc_tiling_on_sc=False)


def _row_split(n):
  rpt = (n // NS) // 8 * 8
  last = n - rpt * (NS - 1)
  assert rpt % 8 == 0 and last % 8 == 0
  return rpt, last


def _deg_kernel(n, ep):
  epc = ep // NC
  ept = epc // NS
  NSUB = CHUNK // SUB
  nch = ept // CHUNK
  assert ept % CHUNK == 0

  @functools.partial(
      pl.kernel, mesh=_mesh,
      out_type=[jax.ShapeDtypeStruct((n,), jnp.float32) for _ in range(NC)],
      compiler_params=_sc_params,
      scratch_types=[
          pltpu.VMEM_SHARED((n,), jnp.float32),
          pltpu.VMEM((NSUB, SUB), jnp.int32),
          pltpu.VMEM((CHUNK,), jnp.float32),
          pltpu.SemaphoreType.DMA,
      ])
  def deg_k(dst2_hbm, ea_hbm, z1_hbm, deg0_hbm, deg1_hbm, deg_sh, d_v, w_v,
            dsem):
    c = lax.axis_index("c")
    s = lax.axis_index("s")

    @pl.when(s == 0)
    def _():
      pltpu.sync_copy(z1_hbm, deg_sh)
    plsc.subcore_barrier()

    base0 = c * epc + s * ept

    def chunk(k, _):
      base = pl.multiple_of(base0 + k * CHUNK, CHUNK)
      pltpu.sync_copy(dst2_hbm.at[pl.ds(pl.multiple_of(base // SUB, 8), NSUB), :], d_v)
      pltpu.sync_copy(ea_hbm.at[pl.ds(base, CHUNK)], w_v)
      sc = [pltpu.async_copy(w_v.at[pl.ds(j * SUB, SUB)],
                             deg_sh.at[d_v.at[j]], dsem, add=True)
            for j in range(NSUB)]
      for dd in sc:
        dd.wait()
      return 0

    lax.fori_loop(0, nch, chunk, 0)
    plsc.subcore_barrier()

    @pl.when(s == 0)
    def _():
      @pl.when(c == 0)
      def _():
        pltpu.sync_copy(deg_sh, deg0_hbm)
      @pl.when(c == 1)
      def _():
        pltpu.sync_copy(deg_sh, deg1_hbm)

  return deg_k


def _norm_kernel(n, ep):
  nw = NC * NS
  ept = ep // nw
  C = 2048
  nch = ept // C
  steps = C // LANES
  assert ept % C == 0

  @functools.partial(
      pl.kernel, mesh=_mesh,
      out_type=jax.ShapeDtypeStruct((ep,), jnp.float32),
      compiler_params=_sc_params,
      scratch_types=[
          pltpu.VMEM((n,), jnp.float32),
          pltpu.VMEM((C,), jnp.int32),
          pltpu.VMEM((C,), jnp.int32),
          pltpu.VMEM((C,), jnp.float32),
          pltpu.VMEM((C,), jnp.float32),
      ])
  def norm_k(src_hbm, dst_hbm, ea_hbm, g_hbm, norm_hbm, g_v, s_v, d_v, w_v, o_v):
    c = lax.axis_index("c")
    s = lax.axis_index("s")
    wid = s * NC + c
    pltpu.sync_copy(g_hbm, g_v)
    base0 = wid * ept

    def chunk(k, _):
      base = pl.multiple_of(base0 + k * C, C)
      pltpu.sync_copy(src_hbm.at[pl.ds(base, C)], s_v)
      pltpu.sync_copy(dst_hbm.at[pl.ds(base, C)], d_v)
      pltpu.sync_copy(ea_hbm.at[pl.ds(base, C)], w_v)

      def step(i, _):
        off = i * LANES
        si = s_v[pl.ds(off, LANES)]
        di = d_v[pl.ds(off, LANES)]
        wv = w_v[pl.ds(off, LANES)]
        o_v[pl.ds(off, LANES)] = (
            plsc.load_gather(g_v, [si]) * wv * plsc.load_gather(g_v, [di]))
        return 0

      lax.fori_loop(0, steps, step, 0)
      pltpu.sync_copy(o_v, norm_hbm.at[pl.ds(base, C)])
      return 0

    lax.fori_loop(0, nch, chunk, 0)

  return norm_k


def _agg_kernel(n, ep, dw):
  ept = ep // NS
  CH = 20480 // dw
  SUBA = 40 if CH // 80 < 8 else 80
  NSUB = CH // SUBA
  nch = ept // CH
  rpt, rlast = _row_split(n)
  assert ept % CH == 0 and nch % 4 == 0 and dw % LANES == 0

  @functools.partial(
      pl.kernel, mesh=_mesh,
      out_type=[jax.ShapeDtypeStruct((n, dw), jnp.float32) for _ in range(NC)],
      compiler_params=_sc_params,
      scratch_types=[
          pltpu.VMEM_SHARED((n, dw), jnp.float32),
          pltpu.VMEM((4, NSUB, SUBA), jnp.int32),
          pltpu.VMEM((4, NSUB, SUBA), jnp.int32),
          pltpu.VMEM((4, CH), jnp.float32),
          pltpu.VMEM((4, CH, dw), jnp.float32),
          pltpu.SemaphoreType.DMA,
          pltpu.SemaphoreType.DMA,
          pltpu.SemaphoreType.DMA,
      ])
  def agg_k(y0_hbm, y1_hbm, src2_hbm, dst2_hbm, nrm_hbm, z_hbm,
            a0_hbm, a1_hbm, acc_sh, s_v, d_v, n_v, rows_v,
            sem_st, sem_g, sem_s):
    c = lax.axis_index("c")
    s = lax.axis_index("s")

    @pl.when(s == 0)
    def _():
      pltpu.sync_copy(z_hbm, acc_sh)
    plsc.subcore_barrier()

    base0 = s * ept

    def base_of(k):
      return pl.multiple_of(base0 + k * CH, CH)

    def stage_copies(k, u, fire):
      base = base_of(k)
      brow = pl.multiple_of(base // SUBA, NSUB)
      f = pltpu.async_copy if fire else pltpu.make_async_copy
      return [f(src2_hbm.at[pl.ds(brow, NSUB), :], s_v.at[u], sem_st),
              f(dst2_hbm.at[pl.ds(brow, NSUB), :], d_v.at[u], sem_st),
              f(nrm_hbm.at[pl.ds(base, CH)], n_v.at[u], sem_st)]

    def issue_stage(k, u):
      stage_copies(k, u, True)

    def wait_stage(k, u):
      for dd in stage_copies(k, u, False):
        dd.wait()

    def gather_copies(y_hbm, u, fire):
      f = pltpu.async_copy if fire else pltpu.make_async_copy
      return [f(y_hbm.at[s_v.at[u, j]],
                rows_v.at[u, pl.ds(j * SUBA, SUBA), :], sem_g)
              for j in range(NSUB)]

    def issue_gather(u):
      @pl.when(c == 0)
      def _():
        gather_copies(y0_hbm, u, True)
      @pl.when(c == 1)
      def _():
        gather_copies(y1_hbm, u, True)

    def wait_gather(u):
      for dd in gather_copies(y0_hbm, u, False):
        dd.wait()

    def scatter_copies(u, fire):
      if fire:
        return [pltpu.async_copy(rows_v.at[u, pl.ds(j * SUBA, SUBA), :],
                                 acc_sh.at[d_v.at[u, j]], sem_s, add=True)
                for j in range(NSUB)]
      return [pltpu.make_async_copy(rows_v.at[u, pl.ds(j * SUBA, SUBA), :],
                                    acc_sh.at[d_v.at[u, j]], sem_s)
              for j in range(NSUB)]

    def issue_scatter(u):
      scatter_copies(u, True)

    def wait_scatter(u):
      for dd in scatter_copies(u, False):
        dd.wait()

    RU = 8

    def scale(u):
      @plsc.parallel_loop(0, CH, step=RU, unroll=2)
      def _(r):
        r0 = pl.multiple_of(r, RU)
        spls = [plsc.load_gather(n_v.at[u],
                                 [jnp.full((LANES,), t, jnp.int32) + r0])
                for t in range(RU)]
        for t in range(RU):
          for jj in range(dw // LANES):
            sl = pl.ds(jj * LANES, LANES)
            rows_v[u, r0 + t, sl] = rows_v[u, r0 + t, sl] * spls[t]

    issue_stage(0, 0)
    issue_stage(1, 1)
    wait_stage(0, 0)
    issue_gather(0)

    def body(m, _):
      for u in range(4):
        k = 4 * m + u
        up1 = (u + 1) % 4
        up2 = (u + 2) % 4

        @pl.when(k >= 2)
        def _():
          wait_scatter(up2)
        @pl.when(k + 2 < nch)
        def _():
          issue_stage(k + 2, up2)
        @pl.when(k + 1 < nch)
        def _():
          wait_stage(k + 1, up1)
          issue_gather(up1)

        wait_gather(u)
        scale(u)
        issue_scatter(u)
      return 0

    lax.fori_loop(0, nch // 4, body, 0)
    wait_scatter((nch - 2) % 4)
    wait_scatter((nch - 1) % 4)
    plsc.subcore_barrier()

    r0 = pl.multiple_of(s * rpt, 8)
    for cc, a_hbm in ((0, a0_hbm), (1, a1_hbm)):
      @pl.when((c == cc) & (s < NS - 1))
      def _():
        pltpu.sync_copy(acc_sh.at[pl.ds(r0, rpt), :],
                        a_hbm.at[pl.ds(r0, rpt), :])
      @pl.when((c == cc) & (s == NS - 1))
      def _():
        pltpu.sync_copy(acc_sh.at[pl.ds(r0, rlast), :],
                        a_hbm.at[pl.ds(r0, rlast), :])

  return agg_k


def _prep_kernel(r, q):
  def body(d0_ref, d1_ref, g_ref, g2_ref):
    deg = d0_ref[...] + d1_ref[...] + 1.0
    g = jnp.where(deg > 0, lax.rsqrt(deg), 0.0)
    g_ref[...] = g
    g2_ref[...] = g * g

  return pl.pallas_call(
      body,
      out_shape=[jax.ShapeDtypeStruct((r, q), jnp.float32)] * 2)


def _mm_first(n, din, dq):
  B = 1000
  dh = dq // 2

  def body(x_ref, w_ref, y0_ref, y1_ref):
    y = jnp.dot(x_ref[...], w_ref[...], preferred_element_type=jnp.float32)
    y0_ref[...] = y[:, :dh]
    y1_ref[...] = y[:, dh:]

  return pl.pallas_call(
      body,
      grid=(n // B,),
      in_specs=[pl.BlockSpec((B, din), lambda i: (i, 0)),
                pl.BlockSpec((din, dq), lambda i: (0, 0))],
      out_specs=[pl.BlockSpec((B, dh), lambda i: (i, 0))] * 2,
      out_shape=[jax.ShapeDtypeStruct((n, dh), jnp.float32)] * 2)


def _mm_mid(n, dp, dq, act):
  B = 1000
  dhp = dp // 2
  dhq = dq // 2

  def body(a0_ref, a1_ref, y0_ref, y1_ref, g2_ref, b_ref, w_ref,
           o0_ref, o1_ref):
    agg = jnp.concatenate([a0_ref[...], a1_ref[...]], axis=1)
    y = jnp.concatenate([y0_ref[...], y1_ref[...]], axis=1)
    t = agg + g2_ref[...] * y + b_ref[...]
    if act:
      t = jnp.where(t > 0, t, jnp.exp(jnp.minimum(t, 0.0)) - 1.0)
    z = jnp.dot(t, w_ref[...], preferred_element_type=jnp.float32)
    o0_ref[...] = z[:, :dhq]
    o1_ref[...] = z[:, dhq:]

  return pl.pallas_call(
      body,
      grid=(n // B,),
      in_specs=[pl.BlockSpec((B, dhp), lambda i: (i, 0)),
                pl.BlockSpec((B, dhp), lambda i: (i, 0)),
                pl.BlockSpec((B, dhp), lambda i: (i, 0)),
                pl.BlockSpec((B, dhp), lambda i: (i, 0)),
                pl.BlockSpec((B, 1), lambda i: (i, 0)),
                pl.BlockSpec((1, dp), lambda i: (0, 0)),
                pl.BlockSpec((dp, dq), lambda i: (0, 0))],
      out_specs=[pl.BlockSpec((B, dhq), lambda i: (i, 0))] * 2,
      out_shape=[jax.ShapeDtypeStruct((n, dhq), jnp.float32)] * 2)


def _mm_final(n, dp):
  B = 1000
  dhp = dp // 2

  def body(a0_ref, a1_ref, y0_ref, y1_ref, g2_ref, b_ref, o_ref):
    agg = jnp.concatenate([a0_ref[...], a1_ref[...]], axis=1)
    y = jnp.concatenate([y0_ref[...], y1_ref[...]], axis=1)
    o_ref[...] = agg + g2_ref[...] * y + b_ref[...]

  return pl.pallas_call(
      body,
      grid=(n // B,),
      in_specs=[pl.BlockSpec((B, dhp), lambda i: (i, 0)),
                pl.BlockSpec((B, dhp), lambda i: (i, 0)),
                pl.BlockSpec((B, dhp), lambda i: (i, 0)),
                pl.BlockSpec((B, dhp), lambda i: (i, 0)),
                pl.BlockSpec((B, 1), lambda i: (i, 0)),
                pl.BlockSpec((1, dp), lambda i: (0, 0))],
      out_specs=pl.BlockSpec((B, dp), lambda i: (i, 0)),
      out_shape=jax.ShapeDtypeStruct((n, dp), jnp.float32))


def kernel(x, edge_index, edge_attr, W1, b1, W2, b2, W3, b3, W4, b4,
           W5, b5, W6, b6):
  n, f_in = x.shape
  e = edge_index.shape[1]

  align = NC * NS * CHUNK
  ep = (e + align - 1) // align * align
  pad = ep - e
  src = jnp.concatenate([edge_index[0], jnp.zeros((pad,), jnp.int32)])
  dst = jnp.concatenate([edge_index[1], jnp.zeros((pad,), jnp.int32)])
  ea = jnp.concatenate([edge_attr, jnp.zeros((pad,), jnp.float32)])
  iviews = {40: (src.reshape(-1, 40), dst.reshape(-1, 40)),
            80: (src.reshape(-1, 80), dst.reshape(-1, 80))}
  src2, dst2 = iviews[80]

  Ws = [W1, W2, W3, W4, W5, W6]
  bs = [b1, b2, b3, b4, b5, b6]
  PD = [32, 32, 32, 64, 128, 128]

  Wp, bp = [], []
  prev = f_in
  for l in range(6):
    W, b = Ws[l], bs[l]
    wpad = jnp.zeros((prev, PD[l]), jnp.float32)
    wpad = wpad.at[:W.shape[0], :W.shape[1]].set(W)
    bpad = jnp.zeros((1, PD[l]), jnp.float32).at[0, :b.shape[0]].set(b)
    Wp.append(wpad)
    bp.append(bpad)
    prev = PD[l]

  z1 = jnp.zeros((n,), jnp.float32)
  zeros = {dw: jnp.zeros((n, dw), jnp.float32) for dw in {16, 32, 64}}

  deg0, deg1 = _deg_kernel(n, ep)(dst2, ea, z1)
  g2d, g22d = _prep_kernel(80, n // 80)(deg0.reshape(80, -1),
                                        deg1.reshape(80, -1))
  g = g2d.reshape(n)
  g2 = g22d.reshape(n, 1)
  nrm = _norm_kernel(n, ep)(src, dst, ea, g)

  y0, y1 = _mm_first(n, f_in, PD[0])(x, Wp[0])
  for l in range(6):
    dw = PD[l] // 2
    suba = 40 if (20480 // dw) // 80 < 8 else 80
    srcv, dstv = iviews[suba]
    a0, a1 = _agg_kernel(n, ep, dw)(y0, y1, srcv, dstv, nrm, zeros[dw])
    if l < 5:
      act = l in (0, 1, 2, 4)
      y0, y1 = _mm_mid(n, PD[l], PD[l + 1], act)(
          a0, a1, y0, y1, g2, bp[l], Wp[l + 1])
    else:
      out = _mm_final(n, PD[5])(a0, a1, y0, y1, g2, bp[5])

  return out.reshape(-1, 128)
